# baseline (speedup 1.0000x reference)
import functools

import jax
import jax.numpy as jnp
from jax import lax
from jax.experimental import pallas as pl
from jax.experimental.pallas import tpu as pltpu
from jax.experimental.pallas import tpu_sc as plsc

_V = 16
_NC = 2
_NS = 16
_NW = _NC * _NS
_N = 10
_A = 5


def _embed_body(tokens_hbm, table_hbm, out_hbm, idx_all, *scratch):
    rows = scratch[:_N]
    gsem = scratch[_N:2 * _N]
    osem = scratch[2 * _N:3 * _N]
    nunits = tokens_hbm.shape[0]
    per_w = nunits // _NW
    wid = lax.axis_index("s") * _NC + lax.axis_index("c")
    base = wid * per_w

    pltpu.sync_copy(tokens_hbm.at[pl.ds(base, per_w)], idx_all)

    def fire_gather(j, b):
        idx_vec = idx_all[j]
        pltpu.async_copy(table_hbm.at[idx_vec], rows[b], gsem[b])

    def drain_gather_fire_write(j, b):
        pltpu.make_async_copy(table_hbm.at[idx_all.at[j]], rows[b], gsem[b]).wait()
        pltpu.async_copy(rows[b], out_hbm.at[base + j], osem[b])

    def drain_write(j, b):
        pltpu.make_async_copy(rows[b], out_hbm.at[base + j], osem[b]).wait()

    for j in range(_N):
        fire_gather(j, j)
    for j in range(_A):
        drain_gather_fire_write(j, j)

    n_outer = (per_w - _N) // _N

    def outer(o, carry):
        i0 = _N + o * _N
        for b in range(_N):
            i = i0 + b
            drain_gather_fire_write(i - _A, (b - _A) % _N)
            drain_write(i - _N, b)
            fire_gather(i, b)
        return carry

    lax.fori_loop(0, n_outer, outer, 0)

    for j in range(per_w - _A, per_w):
        drain_gather_fire_write(j, j % _N)
    for j in range(per_w - _N, per_w):
        drain_write(j, j % _N)


def kernel(tokens, W_E):
    batch, seq = tokens.shape
    d_model = W_E.shape[1]
    n = batch * seq
    nunits = n // _V
    per_w = nunits // _NW
    tokens2d = tokens.reshape(nunits, _V).astype(jnp.int32)

    mesh = plsc.VectorSubcoreMesh(core_axis_name="c", subcore_axis_name="s")
    fn = functools.partial(
        pl.kernel,
        mesh=mesh,
        out_type=jax.ShapeDtypeStruct((nunits, _V, d_model), jnp.float32),
        scratch_types=(
            [pltpu.VMEM((per_w, _V), jnp.int32)]
            + [pltpu.VMEM((_V, d_model), jnp.float32) for _ in range(_N)]
            + [pltpu.SemaphoreType.DMA for _ in range(2 * _N)]
        ),
        compiler_params=pltpu.CompilerParams(use_tc_tiling_on_sc=False),
    )(_embed_body)
    out = fn(tokens2d, W_E)
    return out.reshape(batch, seq, d_model)


# speedup vs baseline: 1.0860x; 1.0860x over previous
import functools

import jax
import jax.numpy as jnp
from jax import lax
from jax.experimental import pallas as pl
from jax.experimental.pallas import tpu as pltpu
from jax.experimental.pallas import tpu_sc as plsc

_L = 128
_NC = 2
_NS = 16
_NW = _NC * _NS
_N = 10
_A = 5


def _embed_body(tokens_hbm, table_hbm, out_hbm, idx_all, *scratch):
    rows = scratch[:_N]
    gsem = scratch[_N:2 * _N]
    osem = scratch[2 * _N:3 * _N]
    nblocks = tokens_hbm.shape[0]
    per_w = nblocks // _NW
    wid = lax.axis_index("s") * _NC + lax.axis_index("c")
    base = wid * per_w

    pltpu.sync_copy(tokens_hbm.at[pl.ds(base, per_w)], idx_all)

    def fire_gather(j, b):
        pltpu.async_copy(table_hbm.at[idx_all.at[j]], rows[b], gsem[b])

    def drain_gather_fire_write(j, b):
        pltpu.make_async_copy(table_hbm.at[idx_all.at[j]], rows[b], gsem[b]).wait()
        pltpu.async_copy(rows[b], out_hbm.at[base + j], osem[b])

    def drain_write(j, b):
        pltpu.make_async_copy(rows[b], out_hbm.at[base + j], osem[b]).wait()

    for j in range(_N):
        fire_gather(j, j)
    for j in range(_A):
        drain_gather_fire_write(j, j)

    n_outer = (per_w - _N) // _N

    def outer(o, carry):
        i0 = _N + o * _N
        for b in range(_N):
            i = i0 + b
            drain_gather_fire_write(i - _A, (b - _A) % _N)
            drain_write(i - _N, b)
            fire_gather(i, b)
        return carry

    lax.fori_loop(0, n_outer, outer, 0)

    for j in range(per_w - _A, per_w):
        drain_gather_fire_write(j, j % _N)
    for j in range(per_w - _N, per_w):
        drain_write(j, j % _N)


def kernel(tokens, W_E):
    batch, seq = tokens.shape
    d_model = W_E.shape[1]
    n = batch * seq
    nblocks = n // _L
    per_w = nblocks // _NW
    tokens2d = tokens.reshape(nblocks, _L).astype(jnp.int32)

    mesh = plsc.VectorSubcoreMesh(core_axis_name="c", subcore_axis_name="s")
    fn = functools.partial(
        pl.kernel,
        mesh=mesh,
        out_type=jax.ShapeDtypeStruct((nblocks, _L, d_model), jnp.float32),
        scratch_types=(
            [pltpu.VMEM((per_w, _L), jnp.int32)]
            + [pltpu.VMEM((_L, d_model), jnp.float32) for _ in range(_N)]
            + [pltpu.SemaphoreType.DMA for _ in range(2 * _N)]
        ),
        compiler_params=pltpu.CompilerParams(use_tc_tiling_on_sc=False),
    )(_embed_body)
    out = fn(tokens2d, W_E)
    return out.reshape(batch, seq, d_model)
